# baseline (speedup 1.0000x reference)
import functools

import jax
import jax.numpy as jnp
from jax import lax
from jax.experimental import pallas as pl
from jax.experimental.pallas import tpu as pltpu
from jax.experimental.pallas import tpu_sc as plsc

B = 16384
L = 100
D = 32
B_TOT = B * L

NC, NS = 2, 16
NW = NC * NS

CHUNK_I = 1024
IDX_PER_W = B_TOT // NW
NCHUNK = IDX_PER_W // CHUNK_I


@functools.partial(
    pl.kernel,
    out_type=jax.ShapeDtypeStruct((B_TOT, D), jnp.float32),
    mesh=plsc.VectorSubcoreMesh(core_axis_name="c", subcore_axis_name="s"),
    compiler_params=pltpu.CompilerParams(use_tc_tiling_on_sc=False),
    scratch_types=[
        pltpu.VMEM((CHUNK_I,), jnp.int32),
        pltpu.VMEM((CHUNK_I, D), jnp.float32),
        pltpu.SemaphoreType.DMA,
    ],
)
def _emb_gather(table_hbm, idx_hbm, out_hbm, idx_v, rows_v, sem):
    wid = lax.axis_index("s") * NC + lax.axis_index("c")
    base = wid * IDX_PER_W

    def body(g, carry):
        off = base + g * CHUNK_I
        pltpu.sync_copy(idx_hbm.at[pl.ds(off, CHUNK_I)], idx_v)
        pltpu.async_copy(table_hbm.at[idx_v], rows_v, sem).wait()
        pltpu.sync_copy(rows_v, out_hbm.at[pl.ds(off, CHUNK_I)])
        return carry

    lax.fori_loop(0, NCHUNK, body, 0)


def kernel(x, table):
    idx = x.reshape(B_TOT).astype(jnp.int32)
    out = _emb_gather(table, idx)
    return out.reshape(B, L, D)


# speedup vs baseline: 2.7868x; 2.7868x over previous
import functools

import jax
import jax.numpy as jnp
from jax import lax
from jax.experimental import pallas as pl
from jax.experimental.pallas import tpu as pltpu
from jax.experimental.pallas import tpu_sc as plsc

B = 16384
L = 100
D = 32

NC, NS = 2, 16
NW = NC * NS
BBLK = B // NW
LANES = 16


@functools.partial(
    pl.kernel,
    out_type=jax.ShapeDtypeStruct((L, 1, D, B), jnp.float32),
    mesh=plsc.VectorSubcoreMesh(core_axis_name="c", subcore_axis_name="s"),
    compiler_params=pltpu.CompilerParams(
        use_tc_tiling_on_sc=False, needs_layout_passes=False
    ),
    scratch_types=[
        pltpu.VMEM((BBLK,), jnp.int32),
        pltpu.VMEM((BBLK, D), jnp.float32),
        pltpu.VMEM((D, BBLK), jnp.float32),
        pltpu.SemaphoreType.DMA,
    ],
)
def _emb_gather_t(table_hbm, xt_hbm, out_hbm, idx_v, r_v, t_v, sem):
    wid = lax.axis_index("s") * NC + lax.axis_index("c")
    b0 = wid * BBLK

    def body(l, carry):
        pltpu.sync_copy(xt_hbm.at[l, pl.ds(b0, BBLK)], idx_v)
        pltpu.async_copy(table_hbm.at[idx_v], r_v, sem).wait()

        def tbody(j, carry2):
            row_idx = lax.iota(jnp.int32, LANES) + j * LANES
            for dd in range(D):
                col_idx = jnp.full((LANES,), dd, jnp.int32)
                t_v[dd, pl.ds(j * LANES, LANES)] = plsc.load_gather(
                    r_v, [row_idx, col_idx]
                )
            return carry2

        lax.fori_loop(0, BBLK // LANES, tbody, 0)
        pltpu.sync_copy(t_v, out_hbm.at[l, 0, :, pl.ds(b0, BBLK)])
        return carry

    lax.fori_loop(0, L, body, 0)


def kernel(x, table):
    xt = x.T.astype(jnp.int32)
    out = _emb_gather_t(table, xt)
    return out.transpose(3, 0, 1, 2)


# speedup vs baseline: 3.0674x; 1.1007x over previous
import functools

import jax
import jax.numpy as jnp
from jax import lax
from jax.experimental import pallas as pl
from jax.experimental.pallas import tpu as pltpu
from jax.experimental.pallas import tpu_sc as plsc

B = 16384
L = 100
D = 32

NC, NS = 2, 16
NW = NC * NS
BBLK = B // NW
LANES = 16


@functools.partial(
    pl.kernel,
    out_type=jax.ShapeDtypeStruct((L, 1, D, B), jnp.float32),
    mesh=plsc.VectorSubcoreMesh(core_axis_name="c", subcore_axis_name="s"),
    compiler_params=pltpu.CompilerParams(
        use_tc_tiling_on_sc=False, needs_layout_passes=False
    ),
    scratch_types=[
        pltpu.VMEM((BBLK,), jnp.int32),
        pltpu.VMEM((BBLK,), jnp.int32),
        pltpu.VMEM((BBLK, D), jnp.float32),
        pltpu.VMEM((BBLK, D), jnp.float32),
        pltpu.VMEM((D, BBLK), jnp.float32),
        pltpu.VMEM((D, BBLK), jnp.float32),
        pltpu.SemaphoreType.DMA,
        pltpu.SemaphoreType.DMA,
        pltpu.SemaphoreType.DMA,
        pltpu.SemaphoreType.DMA,
    ],
)
def _emb_gather_t(
    table_hbm, xt_hbm, out_hbm,
    idx0_v, idx1_v, r0_v, r1_v, t0_v, t1_v,
    semg0, semg1, semw0, semw1,
):
    wid = lax.axis_index("s") * NC + lax.axis_index("c")
    b0 = wid * BBLK

    def transpose(r_v, t_v):
        def tbody(j, carry):
            row_idx = lax.iota(jnp.int32, LANES) + j * LANES
            for dd in range(D):
                col_idx = jnp.full((LANES,), dd, jnp.int32)
                t_v[dd, pl.ds(j * LANES, LANES)] = plsc.load_gather(
                    r_v, [row_idx, col_idx]
                )
            return carry

        lax.fori_loop(0, BBLK // LANES, tbody, 0)

    def gather_start(idx_v, r_v, sem):
        pltpu.async_copy(table_hbm.at[idx_v], r_v, sem)

    def gather_wait(idx_v, r_v, sem):
        pltpu.make_async_copy(table_hbm.at[idx_v], r_v, sem).wait()

    def write_start(t_v, l, sem):
        pltpu.async_copy(t_v, out_hbm.at[l, 0, :, pl.ds(b0, BBLK)], sem)

    def write_wait(t_v, l, sem):
        pltpu.make_async_copy(
            t_v, out_hbm.at[l, 0, :, pl.ds(b0, BBLK)], sem
        ).wait()

    pltpu.sync_copy(xt_hbm.at[0, pl.ds(b0, BBLK)], idx0_v)
    gather_start(idx0_v, r0_v, semg0)

    def body(g, carry):
        l0 = 2 * g
        l1 = l0 + 1

        pltpu.sync_copy(xt_hbm.at[l1, pl.ds(b0, BBLK)], idx1_v)
        gather_start(idx1_v, r1_v, semg1)

        gather_wait(idx0_v, r0_v, semg0)

        @pl.when(g >= 1)
        def _():
            write_wait(t0_v, l0, semw0)

        transpose(r0_v, t0_v)
        write_start(t0_v, l0, semw0)

        @pl.when(g < L // 2 - 1)
        def _():
            pltpu.sync_copy(xt_hbm.at[l0 + 2, pl.ds(b0, BBLK)], idx0_v)
            gather_start(idx0_v, r0_v, semg0)

        gather_wait(idx1_v, r1_v, semg1)

        @pl.when(g >= 1)
        def _():
            write_wait(t1_v, l1, semw1)

        transpose(r1_v, t1_v)
        write_start(t1_v, l1, semw1)
        return carry

    lax.fori_loop(0, L // 2, body, 0)

    write_wait(t0_v, L - 2, semw0)
    write_wait(t1_v, L - 1, semw1)


def kernel(x, table):
    xt = x.T.astype(jnp.int32)
    out = _emb_gather_t(table, xt)
    return out.transpose(3, 0, 1, 2)


# speedup vs baseline: 5.6206x; 1.8324x over previous
import functools

import jax
import jax.numpy as jnp
from jax import lax
from jax.experimental import pallas as pl
from jax.experimental.pallas import tpu as pltpu
from jax.experimental.pallas import tpu_sc as plsc

B = 16384
L = 100
D = 32

NC, NS = 2, 16
NW = NC * NS
BBLK = B // NW
LANES = 16


@functools.partial(
    pl.kernel,
    out_type=jax.ShapeDtypeStruct((L, 1, D, B), jnp.float32),
    mesh=plsc.VectorSubcoreMesh(core_axis_name="c", subcore_axis_name="s"),
    compiler_params=pltpu.CompilerParams(
        use_tc_tiling_on_sc=False, needs_layout_passes=False
    ),
    scratch_types=[
        pltpu.VMEM((L, BBLK), jnp.int32),
        pltpu.VMEM((BBLK, D), jnp.float32),
        pltpu.VMEM((BBLK, D), jnp.float32),
        pltpu.VMEM((D, BBLK + 1), jnp.float32),
        pltpu.VMEM((D, BBLK + 1), jnp.float32),
        pltpu.SemaphoreType.DMA,
        pltpu.SemaphoreType.DMA,
        pltpu.SemaphoreType.DMA,
        pltpu.SemaphoreType.DMA,
    ],
)
def _emb_gather_t(
    table_hbm, xt_hbm, out_hbm,
    idx_v, r0_v, r1_v, t0_v, t1_v,
    semg0, semg1, semw0, semw1,
):
    wid = lax.axis_index("s") * NC + lax.axis_index("c")
    b0 = wid * BBLK

    UNROLL_B = 16

    def transpose(r_v, t_v):
        row_lo = lax.iota(jnp.int32, LANES)
        row_hi = row_lo + LANES
        zeros = jnp.zeros((LANES,), jnp.int32)

        def tbody(i, carry):
            base = i * UNROLL_B
            for k in range(UNROLL_B):
                bcur = base + k
                col = zeros + bcur
                plsc.store_scatter(t_v, [row_lo, col], r_v[bcur, pl.ds(0, LANES)])
                plsc.store_scatter(t_v, [row_hi, col], r_v[bcur, pl.ds(LANES, LANES)])
            return carry

        lax.fori_loop(0, BBLK // UNROLL_B, tbody, 0)

    def gather_start(l, r_v, sem):
        pltpu.async_copy(table_hbm.at[idx_v.at[l]], r_v, sem)

    def gather_wait(l, r_v, sem):
        pltpu.make_async_copy(table_hbm.at[idx_v.at[l]], r_v, sem).wait()

    def write_start(t_v, l, sem):
        pltpu.async_copy(
            t_v.at[:, pl.ds(0, BBLK)], out_hbm.at[l, 0, :, pl.ds(b0, BBLK)], sem
        )

    def write_wait(t_v, l, sem):
        pltpu.make_async_copy(
            t_v.at[:, pl.ds(0, BBLK)], out_hbm.at[l, 0, :, pl.ds(b0, BBLK)], sem
        ).wait()

    pltpu.sync_copy(xt_hbm.at[:, pl.ds(b0, BBLK)], idx_v)
    gather_start(0, r0_v, semg0)

    def body(g, carry):
        l0 = 2 * g
        l1 = l0 + 1

        gather_start(l1, r1_v, semg1)

        gather_wait(l0, r0_v, semg0)

        @pl.when(g >= 1)
        def _():
            write_wait(t0_v, l0, semw0)

        transpose(r0_v, t0_v)
        write_start(t0_v, l0, semw0)

        @pl.when(g < L // 2 - 1)
        def _():
            gather_start(l0 + 2, r0_v, semg0)

        gather_wait(l1, r1_v, semg1)

        @pl.when(g >= 1)
        def _():
            write_wait(t1_v, l1, semw1)

        transpose(r1_v, t1_v)
        write_start(t1_v, l1, semw1)
        return carry

    lax.fori_loop(0, L // 2, body, 0)

    write_wait(t0_v, L - 2, semw0)
    write_wait(t1_v, L - 1, semw1)


def kernel(x, table):
    xt = x.T.astype(jnp.int32)
    out = _emb_gather_t(table, xt)
    return out.transpose(3, 0, 1, 2)
